# Initial kernel scaffold; baseline (speedup 1.0000x reference)
import jax
import jax.numpy as jnp
from jax.experimental import pallas as pl


def kernel(texts, emb_table, lin_w, lin_b):
    raise NotImplementedError("write your pallas kernel here")


# speedup vs baseline: 89.6958x; 89.6958x over previous
import functools

import jax
import jax.numpy as jnp
from jax import lax
from jax.experimental import pallas as pl
from jax.experimental.pallas import tpu as pltpu
from jax.experimental.pallas import tpu_sc as plsc

VOCAB = 30522
EMBED_DIM = 128
NUM_CLASS = 3
BATCH = 16384
SEQ = 200

VPAD = 30720
VBLK = 1024

NC = 2
NS = 16
NW = NC * NS
BPW = BATCH // NW
GPW = BPW // 16


def _proj_body(w_ref, b_ref, emb_ref, out_ref):
    out_ref[...] = lax.dot_general(
        w_ref[...], emb_ref[...], (((1,), (1,)), ((), ())),
        preferred_element_type=jnp.float32) + b_ref[...]


def _project_table(w8, b8, emb_table):
    return pl.pallas_call(
        _proj_body,
        grid=(VPAD // VBLK,),
        in_specs=[
            pl.BlockSpec((8, EMBED_DIM), lambda i: (0, 0)),
            pl.BlockSpec((8, 1), lambda i: (0, 0)),
            pl.BlockSpec((VBLK, EMBED_DIM), lambda i: (i, 0)),
        ],
        out_specs=pl.BlockSpec((8, VBLK), lambda i: (0, i)),
        out_shape=jax.ShapeDtypeStruct((8, VPAD), jnp.float32),
    )(w8, b8, emb_table)


def _sc_body(p_hbm, texts_hbm, out_hbm, t0, t1, t2, idxbuf, o0, o1, o2):
    cid = lax.axis_index("c")
    sid = lax.axis_index("s")
    wid = sid * NC + cid
    base = wid * BPW

    pltpu.sync_copy(p_hbm.at[0], t0)
    pltpu.sync_copy(p_hbm.at[1], t1)
    pltpu.sync_copy(p_hbm.at[2], t2)

    rows = lax.iota(jnp.int32, 16) * SEQ
    zero = jnp.zeros((16,), jnp.float32)

    for g in range(GPW):
        pltpu.sync_copy(
            texts_hbm.at[pl.ds((base + g * 16) * SEQ, 16 * SEQ)], idxbuf)

        def tstep(t, accs):
            a0, a1, a2 = accs
            ix = plsc.load_gather(idxbuf, [rows + t])
            v0 = plsc.load_gather(t0, [ix])
            v1 = plsc.load_gather(t1, [ix])
            v2 = plsc.load_gather(t2, [ix])
            return a0 + v0, a1 + v1, a2 + v2

        a0, a1, a2 = lax.fori_loop(0, SEQ, tstep, (zero, zero, zero))
        o0[pl.ds(g * 16, 16)] = a0
        o1[pl.ds(g * 16, 16)] = a1
        o2[pl.ds(g * 16, 16)] = a2

    pltpu.sync_copy(o0, out_hbm.at[0, wid])
    pltpu.sync_copy(o1, out_hbm.at[1, wid])
    pltpu.sync_copy(o2, out_hbm.at[2, wid])


@functools.cache
def _sc_gather():
    return pl.kernel(
        _sc_body,
        out_type=jax.ShapeDtypeStruct((NUM_CLASS, NW, BPW), jnp.float32),
        mesh=plsc.VectorSubcoreMesh(core_axis_name="c", subcore_axis_name="s",
                                    num_cores=NC, num_subcores=NS),
        compiler_params=pltpu.CompilerParams(needs_layout_passes=False),
        scratch_types=[
            pltpu.VMEM((VPAD,), jnp.float32),
            pltpu.VMEM((VPAD,), jnp.float32),
            pltpu.VMEM((VPAD,), jnp.float32),
            pltpu.VMEM((16 * SEQ,), jnp.int32),
            pltpu.VMEM((BPW,), jnp.float32),
            pltpu.VMEM((BPW,), jnp.float32),
            pltpu.VMEM((BPW,), jnp.float32),
        ],
    )


def kernel(texts, emb_table, lin_w, lin_b):
    scale = jnp.float32(1.0 / SEQ)
    w8 = jnp.zeros((8, EMBED_DIM), jnp.float32).at[:NUM_CLASS].set(lin_w) * scale
    b8 = jnp.zeros((8, 1), jnp.float32).at[:NUM_CLASS, 0].set(lin_b * scale)
    p = _project_table(w8, b8, emb_table)
    out = _sc_gather()(p, texts.reshape(-1))
    return out.reshape(NUM_CLASS, BATCH).T


# speedup vs baseline: 111.1284x; 1.2389x over previous
import functools

import jax
import jax.numpy as jnp
from jax import lax
from jax.experimental import pallas as pl
from jax.experimental.pallas import tpu as pltpu
from jax.experimental.pallas import tpu_sc as plsc

VOCAB = 30522
EMBED_DIM = 128
NUM_CLASS = 3
BATCH = 16384
SEQ = 200

VPAD = 30720
VBLK = 1024

NC = 2
NS = 16
NW = NC * NS
BPW = BATCH // NW
GRP = 64
NGRP = BPW // GRP
LGRP = GRP // 16
UNROLL = 8


def _proj_body(w_ref, b_ref, emb_ref, out_ref):
    out_ref[...] = lax.dot_general(
        w_ref[...], emb_ref[...], (((1,), (1,)), ((), ())),
        preferred_element_type=jnp.float32) + b_ref[...]


def _project_table(w8, b8, emb_table):
    return pl.pallas_call(
        _proj_body,
        grid=(VPAD // VBLK,),
        in_specs=[
            pl.BlockSpec((8, EMBED_DIM), lambda i: (0, 0)),
            pl.BlockSpec((8, 1), lambda i: (0, 0)),
            pl.BlockSpec((VBLK, EMBED_DIM), lambda i: (i, 0)),
        ],
        out_specs=pl.BlockSpec((8, VBLK), lambda i: (0, i)),
        out_shape=jax.ShapeDtypeStruct((8, VPAD), jnp.float32),
    )(w8, b8, emb_table)


def _sc_body(p_hbm, texts_hbm, out_hbm,
             t0, t1, t2, ib0, ib1, ov, s0, s1, st):
    cid = lax.axis_index("c")
    sid = lax.axis_index("s")
    wid = sid * NC + cid
    base = wid * BPW * SEQ

    ibufs = (ib0, ib1)
    sems = (s0, s1)

    def fetch(g):
        return pltpu.async_copy(
            texts_hbm.at[pl.ds(base + g * GRP * SEQ, GRP * SEQ)],
            ibufs[g % 2], sems[g % 2])

    pending = fetch(0)
    pltpu.async_copy(p_hbm.at[0], t0, st).wait()
    pltpu.async_copy(p_hbm.at[1], t1, st).wait()
    pltpu.async_copy(p_hbm.at[2], t2, st).wait()

    iota = lax.iota(jnp.int32, 16)
    zero = jnp.zeros((16,), jnp.float32)

    for g in range(NGRP):
        nxt = fetch(g + 1) if g + 1 < NGRP else None
        pending.wait()
        pending = nxt
        ib = ibufs[g % 2]

        for l in range(LGRP):
            rows = (iota + l * 16) * SEQ

            def tstep(j, accs):
                a0, a1, a2 = accs
                t = j * UNROLL
                for u in range(UNROLL):
                    ix = plsc.load_gather(ib, [rows + (t + u)])
                    a0 = a0 + plsc.load_gather(t0, [ix])
                    a1 = a1 + plsc.load_gather(t1, [ix])
                    a2 = a2 + plsc.load_gather(t2, [ix])
                return a0, a1, a2

            a0, a1, a2 = lax.fori_loop(
                0, SEQ // UNROLL, tstep, (zero, zero, zero))

            lanes = (g * GRP + l * 16 + iota) * NUM_CLASS
            plsc.store_scatter(ov, [lanes], a0)
            plsc.store_scatter(ov, [lanes + 1], a1)
            plsc.store_scatter(ov, [lanes + 2], a2)

    pltpu.sync_copy(ov, out_hbm.at[pl.ds(wid * BPW * NUM_CLASS,
                                         BPW * NUM_CLASS)])


@functools.cache
def _sc_gather():
    return pl.kernel(
        _sc_body,
        out_type=jax.ShapeDtypeStruct((BATCH * NUM_CLASS,), jnp.float32),
        mesh=plsc.VectorSubcoreMesh(core_axis_name="c", subcore_axis_name="s",
                                    num_cores=NC, num_subcores=NS),
        compiler_params=pltpu.CompilerParams(needs_layout_passes=False),
        scratch_types=[
            pltpu.VMEM((VPAD,), jnp.float32),
            pltpu.VMEM((VPAD,), jnp.float32),
            pltpu.VMEM((VPAD,), jnp.float32),
            pltpu.VMEM((GRP * SEQ,), jnp.int32),
            pltpu.VMEM((GRP * SEQ,), jnp.int32),
            pltpu.VMEM((BPW * NUM_CLASS,), jnp.float32),
            pltpu.SemaphoreType.DMA,
            pltpu.SemaphoreType.DMA,
            pltpu.SemaphoreType.DMA,
        ],
    )


def kernel(texts, emb_table, lin_w, lin_b):
    scale = jnp.float32(1.0 / SEQ)
    w8 = jnp.zeros((8, EMBED_DIM), jnp.float32).at[:NUM_CLASS].set(lin_w) * scale
    b8 = jnp.zeros((8, 1), jnp.float32).at[:NUM_CLASS, 0].set(lin_b * scale)
    p = _project_table(w8, b8, emb_table)
    out = _sc_gather()(p, texts.reshape(-1))
    return out.reshape(BATCH, NUM_CLASS)
